# Initial kernel scaffold; baseline (speedup 1.0000x reference)
import jax
import jax.numpy as jnp
from jax.experimental import pallas as pl


def kernel(x, g1_fc1_w, g1_fc1_b, g1_bn1_g, g1_bn1_b, g1_rel, g1_mr_w, g1_mr_b, g1_fc2_w, g1_fc2_b, g1_bn2_g, g1_bn2_b, g2_fc1_w, g2_fc1_b, g2_bn1_g, g2_bn1_b, g2_rel, g2_mr_w, g2_mr_b, g2_fc2_w, g2_fc2_b, g2_bn2_g, g2_bn2_b, s1_gn_w, s1_gn_b, s1_sq1_w, s1_sq2_w, s1_gwc_w, s1_gwc_b, s1_pwc1_w, s1_pwc2_w, s2_gn_w, s2_gn_b, s2_sq1_w, s2_sq2_w, s2_gwc_w, s2_gwc_b, s2_pwc1_w, s2_pwc2_w):
    raise NotImplementedError("write your pallas kernel here")


# speedup vs baseline: 202.1040x; 202.1040x over previous
import jax
import jax.numpy as jnp
from jax.experimental import pallas as pl
from jax.experimental.pallas import tpu as pltpu

_B, _C, _H, _W = 16, 96, 32, 32
_N = _H * _W
_K = 7
_C2 = 2 * _C
_EPS_BN = 1e-5
_EPS_IN = 1e-5
_EPS_GN = 1e-10


def _mm(a, b):
    return jax.lax.dot_general(a.astype(jnp.bfloat16), b.astype(jnp.bfloat16),
                               (((1,), (0,)), ((), ())),
                               preferred_element_type=jnp.float32)


def _fc1_body(x_ref, w_ref, bcol_ref, y_ref, st_ref):
    b = pl.program_id(0)
    y = _mm(w_ref[...], x_ref[0]) + bcol_ref[...]
    y_ref[0] = y

    @pl.when(b == 0)
    def _():
        st_ref[...] = jnp.zeros_like(st_ref)

    st_ref[0, :, 0:1] = st_ref[0, :, 0:1] + jnp.sum(y, axis=1, keepdims=True)
    st_ref[0, :, 1:2] = st_ref[0, :, 1:2] + jnp.sum(y * y, axis=1, keepdims=True)


def _fc1_call(x, w, bcol):
    return pl.pallas_call(
        _fc1_body,
        grid=(_B,),
        in_specs=[
            pl.BlockSpec((1, _C, _N), lambda b: (b, 0, 0)),
            pl.BlockSpec((_C, _C), lambda b: (0, 0)),
            pl.BlockSpec((_C, 1), lambda b: (0, 0)),
        ],
        out_specs=[
            pl.BlockSpec((1, _C, _N), lambda b: (b, 0, 0)),
            pl.BlockSpec((1, _C, 8), lambda b: (0, 0, 0)),
        ],
        out_shape=[
            jax.ShapeDtypeStruct((_B, _C, _N), jnp.float32),
            jax.ShapeDtypeStruct((1, _C, 8), jnp.float32),
        ],
    )(x, w, bcol)


def _grapher_body(y_ref, bm_ref, bd_ref, bg_ref, bb_ref, rel_ref, wa_ref,
                  wb_ref, mb_ref, w2_ref, b2_ref, out_ref, st_ref):
    b = pl.program_id(0)
    y = (y_ref[0] - bm_ref[...]) / bd_ref[...] * bg_ref[...] + bb_ref[...]
    xn = y / jnp.maximum(
        jnp.sqrt(jnp.sum(y * y, axis=0, keepdims=True)), 1e-12)
    xnt = jnp.transpose(xn)
    sq_col = jnp.sum(xnt * xnt, axis=1, keepdims=True)
    sq_row = jnp.sum(xn * xn, axis=0, keepdims=True)
    cur = sq_col - 2.0 * _mm(xnt, xn) + sq_row + rel_ref[...]
    iota_l = jax.lax.broadcasted_iota(jnp.int32, (_N, _N), 1)
    dmax = jnp.full((_C, _N), -jnp.inf, dtype=jnp.float32)
    for _ in range(_K):
        minv = jnp.min(cur, axis=1, keepdims=True)
        idx = jnp.min(jnp.where(cur == minv, iota_l, _N), axis=1, keepdims=True)
        sel = iota_l == idx
        cur = jnp.where(sel, jnp.inf, cur)
        oh = jnp.where(sel, 1.0, 0.0)
        g = jax.lax.dot_general(y, oh, (((1,), (1,)), ((), ())),
                                preferred_element_type=jnp.float32,
                                precision=jax.lax.Precision.HIGHEST)
        dmax = jnp.maximum(dmax, g)
    d = dmax - y
    m = _mm(wa_ref[...], y) + _mm(wb_ref[...], d) + mb_ref[...]
    mu = jnp.mean(m, axis=1, keepdims=True)
    va = jnp.mean((m - mu) ** 2, axis=1, keepdims=True)
    m = jax.nn.gelu((m - mu) / jnp.sqrt(va + _EPS_IN))
    out = _mm(w2_ref[...], m) + b2_ref[...]
    out_ref[0] = out

    @pl.when(b == 0)
    def _():
        st_ref[...] = jnp.zeros_like(st_ref)

    st_ref[0, :, 0:1] = st_ref[0, :, 0:1] + jnp.sum(out, axis=1, keepdims=True)
    st_ref[0, :, 1:2] = st_ref[0, :, 1:2] + jnp.sum(out * out, axis=1,
                                                    keepdims=True)


def _grapher_call(y_raw, bn_cols, rel, wa, wb, mb, w2, b2):
    return pl.pallas_call(
        _grapher_body,
        grid=(_B,),
        in_specs=[
            pl.BlockSpec((1, _C, _N), lambda b: (b, 0, 0)),
            pl.BlockSpec((_C, 1), lambda b: (0, 0)),
            pl.BlockSpec((_C, 1), lambda b: (0, 0)),
            pl.BlockSpec((_C, 1), lambda b: (0, 0)),
            pl.BlockSpec((_C, 1), lambda b: (0, 0)),
            pl.BlockSpec((_N, _N), lambda b: (0, 0)),
            pl.BlockSpec((_C2, _C), lambda b: (0, 0)),
            pl.BlockSpec((_C2, _C), lambda b: (0, 0)),
            pl.BlockSpec((_C2, 1), lambda b: (0, 0)),
            pl.BlockSpec((_C, _C2), lambda b: (0, 0)),
            pl.BlockSpec((_C, 1), lambda b: (0, 0)),
        ],
        out_specs=[
            pl.BlockSpec((1, _C, _N), lambda b: (b, 0, 0)),
            pl.BlockSpec((1, _C, 8), lambda b: (0, 0, 0)),
        ],
        out_shape=[
            jax.ShapeDtypeStruct((_B, _C, _N), jnp.float32),
            jax.ShapeDtypeStruct((1, _C, 8), jnp.float32),
        ],
    )(y_raw, *bn_cols, rel, wa, wb, mb, w2, b2)


def _scconv(xx, gnw, gnb, wg, sq1, sq2, w9_ref, gwcb, pwc1, pwc2, pad_ref,
            wmod):
    blocks = []
    for g in range(4):
        blk = xx[24 * g:24 * (g + 1), :]
        mu = jnp.mean(blk)
        sd = jnp.sqrt(jnp.mean((blk - mu) ** 2))
        blocks.append((blk - mu) / (sd + _EPS_GN))
    gx = jnp.concatenate(blocks, axis=0) * gnw + gnb
    info = jax.nn.sigmoid(gx * wg) >= 0.5
    x1 = jnp.where(info, gx, 0.0)
    x2 = gx - x1
    h = _C // 2
    up = _mm(sq1, x1[:h] + x2[h:])
    low = _mm(sq2, x1[h:] + x2[:h])
    pad_ref[...] = jnp.zeros_like(pad_ref)
    pad_ref[:, 33:33 + _N] = up
    acc = jnp.zeros((_C, _N), jnp.float32)
    k = 0
    for dy in (-1, 0, 1):
        for dx in (-1, 0, 1):
            off = 33 + 32 * dy + dx
            xs = pad_ref[:, off:off + _N]
            if dx == -1:
                xs = jnp.where(wmod != 0, xs, 0.0)
            elif dx == 1:
                xs = jnp.where(wmod != 31, xs, 0.0)
            acc = acc + _mm(w9_ref[k], xs)
            k += 1
    y1c = acc + gwcb + _mm(pwc1, up)
    y2c = jnp.concatenate([_mm(pwc2, low), low], axis=0)
    mcat = jnp.concatenate([jnp.mean(y1c, axis=1, keepdims=True),
                            jnp.mean(y2c, axis=1, keepdims=True)], axis=0)
    e = jnp.exp(mcat - jnp.max(mcat))
    att = e / jnp.sum(e)
    return att[:_C] * y1c + att[_C:] * y2c


def _post_fc1_body(in_ref, bm_ref, bd_ref, bg_ref, bb_ref, res_ref, gnw_ref,
                   gnb_ref, wg_ref, sq1_ref, sq2_ref, w9_ref, gwcb_ref,
                   pwc1_ref, pwc2_ref, f1w_ref, f1b_ref, t_ref, y2_ref,
                   st_ref, pad_ref):
    b = pl.program_id(0)
    wmod = jax.lax.broadcasted_iota(jnp.int32, (1, _N), 1) % 32
    xx = ((in_ref[0] - bm_ref[...]) / bd_ref[...] * bg_ref[...]
          + bb_ref[...] + res_ref[0])
    so = _scconv(xx, gnw_ref[...], gnb_ref[...], wg_ref[...], sq1_ref[...],
                 sq2_ref[...], w9_ref, gwcb_ref[...], pwc1_ref[...],
                 pwc2_ref[...], pad_ref, wmod)
    mu = jnp.mean(so, axis=1, keepdims=True)
    va = jnp.mean((so - mu) ** 2, axis=1, keepdims=True)
    t = jnp.maximum((so - mu) / jnp.sqrt(va + _EPS_IN), 0.0)
    t_ref[0] = t
    y2 = _mm(f1w_ref[...], t) + f1b_ref[...]
    y2_ref[0] = y2

    @pl.when(b == 0)
    def _():
        st_ref[...] = jnp.zeros_like(st_ref)

    st_ref[0, :, 0:1] = st_ref[0, :, 0:1] + jnp.sum(y2, axis=1, keepdims=True)
    st_ref[0, :, 1:2] = st_ref[0, :, 1:2] + jnp.sum(y2 * y2, axis=1,
                                                    keepdims=True)


def _post_fc1_call(in_raw, bn_cols, res, gnw, gnb, wg, sq1, sq2, w9, gwcb,
                   pwc1, pwc2, f1w, f1b):
    small = lambda shape: pl.BlockSpec(shape, lambda b: (0,) * len(shape))
    return pl.pallas_call(
        _post_fc1_body,
        grid=(_B,),
        in_specs=[
            pl.BlockSpec((1, _C, _N), lambda b: (b, 0, 0)),
            small((_C, 1)), small((_C, 1)), small((_C, 1)), small((_C, 1)),
            pl.BlockSpec((1, _C, _N), lambda b: (b, 0, 0)),
            small((_C, 1)), small((_C, 1)), small((_C, 1)),
            small((24, 48)), small((24, 48)),
            small((9, _C, 24)), small((_C, 1)),
            small((_C, 24)), small((_C - 24, 24)),
            small((_C, _C)), small((_C, 1)),
        ],
        out_specs=[
            pl.BlockSpec((1, _C, _N), lambda b: (b, 0, 0)),
            pl.BlockSpec((1, _C, _N), lambda b: (b, 0, 0)),
            pl.BlockSpec((1, _C, 8), lambda b: (0, 0, 0)),
        ],
        out_shape=[
            jax.ShapeDtypeStruct((_B, _C, _N), jnp.float32),
            jax.ShapeDtypeStruct((_B, _C, _N), jnp.float32),
            jax.ShapeDtypeStruct((1, _C, 8), jnp.float32),
        ],
        scratch_shapes=[pltpu.VMEM((24, 1152), jnp.float32)],
    )(in_raw, *bn_cols, res, gnw, gnb, wg, sq1, sq2, w9, gwcb, pwc1, pwc2,
      f1w, f1b)


def _post_final_body(in_ref, bm_ref, bd_ref, bg_ref, bb_ref, res_ref,
                     gnw_ref, gnb_ref, wg_ref, sq1_ref, sq2_ref, w9_ref,
                     gwcb_ref, pwc1_ref, pwc2_ref, x_ref, fin_ref, pad_ref):
    wmod = jax.lax.broadcasted_iota(jnp.int32, (1, _N), 1) % 32
    xx = ((in_ref[0] - bm_ref[...]) / bd_ref[...] * bg_ref[...]
          + bb_ref[...] + res_ref[0])
    so = _scconv(xx, gnw_ref[...], gnb_ref[...], wg_ref[...], sq1_ref[...],
                 sq2_ref[...], w9_ref, gwcb_ref[...], pwc1_ref[...],
                 pwc2_ref[...], pad_ref, wmod)
    mu = jnp.mean(so, axis=1, keepdims=True)
    va = jnp.mean((so - mu) ** 2, axis=1, keepdims=True)
    fin_ref[0] = (so - mu) / jnp.sqrt(va + _EPS_IN) + x_ref[0]


def _post_final_call(in_raw, bn_cols, res, gnw, gnb, wg, sq1, sq2, w9, gwcb,
                     pwc1, pwc2, x):
    small = lambda shape: pl.BlockSpec(shape, lambda b: (0,) * len(shape))
    return pl.pallas_call(
        _post_final_body,
        grid=(_B,),
        in_specs=[
            pl.BlockSpec((1, _C, _N), lambda b: (b, 0, 0)),
            small((_C, 1)), small((_C, 1)), small((_C, 1)), small((_C, 1)),
            pl.BlockSpec((1, _C, _N), lambda b: (b, 0, 0)),
            small((_C, 1)), small((_C, 1)), small((_C, 1)),
            small((24, 48)), small((24, 48)),
            small((9, _C, 24)), small((_C, 1)),
            small((_C, 24)), small((_C - 24, 24)),
            pl.BlockSpec((1, _C, _N), lambda b: (b, 0, 0)),
        ],
        out_specs=[pl.BlockSpec((1, _C, _N), lambda b: (b, 0, 0))],
        out_shape=[jax.ShapeDtypeStruct((_B, _C, _N), jnp.float32)],
        scratch_shapes=[pltpu.VMEM((24, 1152), jnp.float32)],
    )(in_raw, *bn_cols, res, gnw, gnb, wg, sq1, sq2, w9, gwcb, pwc1, pwc2, x)


def _bn_cols(st, g, b):
    cnt = float(_B * _N)
    mean = st[0, :, 0] / cnt
    var = st[0, :, 1] / cnt - mean * mean
    den = jnp.sqrt(var + _EPS_BN)
    return mean[:, None], den[:, None], g[:, None], b[:, None]


def _mr_split(mw):
    gi = jnp.arange(_C2) // 48

    def pick(j):
        valid = (j[None, :] // 48) == gi[:, None]
        q = jnp.clip(j[None, :] - 48 * gi[:, None], 0, 47)
        return jnp.where(valid, jnp.take_along_axis(mw, q, axis=1), 0.0)

    je = 2 * jnp.arange(_C)
    return pick(je), pick(je + 1)


def _gwc9(wc):
    wt = jnp.transpose(wc, (2, 3, 0, 1))
    z = jnp.zeros((3, 3, 48, 12), jnp.float32)
    top = jnp.concatenate([wt[:, :, :48, :], z], axis=3)
    bot = jnp.concatenate([z, wt[:, :, 48:, :]], axis=3)
    return jnp.concatenate([top, bot], axis=2).reshape(9, _C, 24)


def kernel(x, g1_fc1_w, g1_fc1_b, g1_bn1_g, g1_bn1_b, g1_rel, g1_mr_w,
           g1_mr_b, g1_fc2_w, g1_fc2_b, g1_bn2_g, g1_bn2_b,
           g2_fc1_w, g2_fc1_b, g2_bn1_g, g2_bn1_b, g2_rel, g2_mr_w,
           g2_mr_b, g2_fc2_w, g2_fc2_b, g2_bn2_g, g2_bn2_b,
           s1_gn_w, s1_gn_b, s1_sq1_w, s1_sq2_w, s1_gwc_w, s1_gwc_b,
           s1_pwc1_w, s1_pwc2_w,
           s2_gn_w, s2_gn_b, s2_sq1_w, s2_sq2_w, s2_gwc_w, s2_gwc_b,
           s2_pwc1_w, s2_pwc2_w):
    col = lambda v: v.reshape(-1, 1)
    xf = x.reshape(_B, _C, _N)

    y1r, st1 = _fc1_call(xf, g1_fc1_w, col(g1_fc1_b))
    bn1 = _bn_cols(st1, g1_bn1_g, g1_bn1_b)
    wa1, wb1 = _mr_split(g1_mr_w)
    o2r, st2 = _grapher_call(y1r, bn1, g1_rel.reshape(_N, _N), wa1, wb1,
                             col(g1_mr_b), g1_fc2_w, col(g1_fc2_b))
    bn2 = _bn_cols(st2, g1_bn2_g, g1_bn2_b)

    t, y2r, st3 = _post_fc1_call(
        o2r, bn2, xf, col(s1_gn_w), col(s1_gn_b),
        col(s1_gn_w / jnp.sum(s1_gn_w)), s1_sq1_w, s1_sq2_w, _gwc9(s1_gwc_w),
        col(s1_gwc_b), s1_pwc1_w, s1_pwc2_w, g2_fc1_w, col(g2_fc1_b))
    bn3 = _bn_cols(st3, g2_bn1_g, g2_bn1_b)
    wa2, wb2 = _mr_split(g2_mr_w)
    o4r, st4 = _grapher_call(y2r, bn3, g2_rel.reshape(_N, _N), wa2, wb2,
                             col(g2_mr_b), g2_fc2_w, col(g2_fc2_b))
    bn4 = _bn_cols(st4, g2_bn2_g, g2_bn2_b)

    fin = _post_final_call(
        o4r, bn4, t, col(s2_gn_w), col(s2_gn_b),
        col(s2_gn_w / jnp.sum(s2_gn_w)), s2_sq1_w, s2_sq2_w, _gwc9(s2_gwc_w),
        col(s2_gwc_b), s2_pwc1_w, s2_pwc2_w, xf)
    return fin[0].reshape(_B, _C, _H, _W)


# speedup vs baseline: 268.2444x; 1.3273x over previous
import jax
import jax.numpy as jnp
from jax.experimental import pallas as pl
from jax.experimental.pallas import tpu as pltpu

_B, _C, _H, _W = 16, 96, 32, 32
_N = _H * _W
_K = 7
_C2 = 2 * _C
_EPS_BN = 1e-5
_EPS_IN = 1e-5
_EPS_GN = 1e-10


def _mm(a, b):
    return jax.lax.dot_general(a.astype(jnp.bfloat16), b.astype(jnp.bfloat16),
                               (((1,), (0,)), ((), ())),
                               preferred_element_type=jnp.float32)


def _fc1_body(x_ref, w_ref, bcol_ref, y_ref, acc_ref, st_ref):
    b = pl.program_id(0)
    y = _mm(w_ref[...], x_ref[0]) + bcol_ref[...]
    y_ref[0] = y

    @pl.when(b == 0)
    def _():
        acc_ref[...] = jnp.zeros_like(acc_ref)

    a = acc_ref[0, :, 0:128]
    for k in range(_N // 128):
        a = a + y[:, 128 * k:128 * (k + 1)]
    acc_ref[0, :, 0:128] = a

    @pl.when(b == _B - 1)
    def _():
        st_ref[...] = jnp.zeros_like(st_ref)
        st_ref[0, :, 0:1] = jnp.sum(acc_ref[0, :, 0:128], axis=1,
                                    keepdims=True)


def _fc1_call(x, w, bcol):
    return pl.pallas_call(
        _fc1_body,
        grid=(_B,),
        in_specs=[
            pl.BlockSpec((1, _C, _N), lambda b: (b, 0, 0)),
            pl.BlockSpec((_C, _C), lambda b: (0, 0)),
            pl.BlockSpec((_C, 1), lambda b: (0, 0)),
        ],
        out_specs=[
            pl.BlockSpec((1, _C, _N), lambda b: (b, 0, 0)),
            pl.BlockSpec((1, _C, _N), lambda b: (0, 0, 0)),
            pl.BlockSpec((1, _C, 8), lambda b: (0, 0, 0)),
        ],
        out_shape=[
            jax.ShapeDtypeStruct((_B, _C, _N), jnp.float32),
            jax.ShapeDtypeStruct((1, _C, _N), jnp.float32),
            jax.ShapeDtypeStruct((1, _C, 8), jnp.float32),
        ],
    )(x, w, bcol)


def _var_body(y_ref, m_ref, acc_ref, st_ref):
    b = pl.program_id(0)
    t = y_ref[0] - m_ref[...]
    t = t * t

    @pl.when(b == 0)
    def _():
        acc_ref[...] = jnp.zeros_like(acc_ref)

    a = acc_ref[0, :, 0:128]
    for k in range(_N // 128):
        a = a + t[:, 128 * k:128 * (k + 1)]
    acc_ref[0, :, 0:128] = a

    @pl.when(b == _B - 1)
    def _():
        st_ref[...] = jnp.zeros_like(st_ref)
        st_ref[0, :, 0:1] = jnp.sum(acc_ref[0, :, 0:128], axis=1,
                                    keepdims=True)


def _var_call(y, mcol):
    return pl.pallas_call(
        _var_body,
        grid=(_B,),
        in_specs=[
            pl.BlockSpec((1, _C, _N), lambda b: (b, 0, 0)),
            pl.BlockSpec((_C, 1), lambda b: (0, 0)),
        ],
        out_specs=[
            pl.BlockSpec((1, _C, _N), lambda b: (0, 0, 0)),
            pl.BlockSpec((1, _C, 8), lambda b: (0, 0, 0)),
        ],
        out_shape=[
            jax.ShapeDtypeStruct((1, _C, _N), jnp.float32),
            jax.ShapeDtypeStruct((1, _C, 8), jnp.float32),
        ],
    )(y, mcol)[0]


def _grapher_body(y_ref, bm_ref, bd_ref, bg_ref, bb_ref, rel_ref, wa_ref,
                  wb_ref, mb_ref, w2_ref, b2_ref, out_ref, st_ref):
    b = pl.program_id(0)
    y = ((y_ref[0] - bm_ref[...]) * jax.lax.rsqrt(bd_ref[...] + _EPS_BN)
         * bg_ref[...] + bb_ref[...])
    xn = y / jnp.maximum(
        jnp.sqrt(jnp.sum(y * y, axis=0, keepdims=True)), 1e-12)
    xnt = jnp.transpose(xn)
    sq_col = jnp.sum(xnt * xnt, axis=1, keepdims=True)
    sq_row = jnp.sum(xn * xn, axis=0, keepdims=True)
    cur = sq_col - 2.0 * _mm(xnt, xn) + sq_row + rel_ref[...]
    iota_l = jax.lax.broadcasted_iota(jnp.int32, (_N, _N), 1)
    y1s = y.astype(jnp.bfloat16)
    r1 = y - y1s.astype(jnp.float32)
    y2s = r1.astype(jnp.bfloat16)
    y3s = (r1 - y2s.astype(jnp.float32)).astype(jnp.bfloat16)
    dmax = jnp.full((_C, _N), -jnp.inf, dtype=jnp.float32)
    for _ in range(_K):
        minv = jnp.min(cur, axis=1, keepdims=True)
        idx = jnp.min(jnp.where(cur == minv, iota_l, _N), axis=1, keepdims=True)
        sel = iota_l == idx
        cur = jnp.where(sel, jnp.inf, cur)
        oh = jnp.where(sel, 1.0, 0.0).astype(jnp.bfloat16)
        nt = (((1,), (1,)), ((), ()))
        g = (jax.lax.dot_general(y1s, oh, nt, preferred_element_type=jnp.float32)
             + jax.lax.dot_general(y2s, oh, nt, preferred_element_type=jnp.float32)
             + jax.lax.dot_general(y3s, oh, nt, preferred_element_type=jnp.float32))
        dmax = jnp.maximum(dmax, g)
    d = dmax - y
    m = _mm(wa_ref[...], y) + _mm(wb_ref[...], d) + mb_ref[...]
    mu = jnp.mean(m, axis=1, keepdims=True)
    va = jnp.mean((m - mu) ** 2, axis=1, keepdims=True)
    m = jax.nn.gelu((m - mu) * jax.lax.rsqrt(va + _EPS_IN))
    out = _mm(w2_ref[...], m) + b2_ref[...]
    out_ref[0] = out

    @pl.when(b == 0)
    def _():
        st_ref[...] = jnp.zeros_like(st_ref)

    st_ref[0, :, 0:1] = st_ref[0, :, 0:1] + jnp.sum(out, axis=1, keepdims=True)
    st_ref[0, :, 1:2] = st_ref[0, :, 1:2] + jnp.sum(out * out, axis=1,
                                                    keepdims=True)


def _grapher_call(y_raw, bn_cols, rel, wa, wb, mb, w2, b2):
    return pl.pallas_call(
        _grapher_body,
        grid=(_B,),
        in_specs=[
            pl.BlockSpec((1, _C, _N), lambda b: (b, 0, 0)),
            pl.BlockSpec((_C, 1), lambda b: (0, 0)),
            pl.BlockSpec((_C, 1), lambda b: (0, 0)),
            pl.BlockSpec((_C, 1), lambda b: (0, 0)),
            pl.BlockSpec((_C, 1), lambda b: (0, 0)),
            pl.BlockSpec((_N, _N), lambda b: (0, 0)),
            pl.BlockSpec((_C2, _C), lambda b: (0, 0)),
            pl.BlockSpec((_C2, _C), lambda b: (0, 0)),
            pl.BlockSpec((_C2, 1), lambda b: (0, 0)),
            pl.BlockSpec((_C, _C2), lambda b: (0, 0)),
            pl.BlockSpec((_C, 1), lambda b: (0, 0)),
        ],
        out_specs=[
            pl.BlockSpec((1, _C, _N), lambda b: (b, 0, 0)),
            pl.BlockSpec((1, _C, 8), lambda b: (0, 0, 0)),
        ],
        out_shape=[
            jax.ShapeDtypeStruct((_B, _C, _N), jnp.float32),
            jax.ShapeDtypeStruct((1, _C, 8), jnp.float32),
        ],
    )(y_raw, *bn_cols, rel, wa, wb, mb, w2, b2)


def _scconv(xx, gnw, gnb, wg, sq1, sq2, w9_ref, gwcb, pwc1, pwc2, pad_ref,
            wmod):
    blocks = []
    for g in range(4):
        blk = xx[24 * g:24 * (g + 1), :]
        mu = jnp.mean(blk)
        sd = jnp.sqrt(jnp.mean((blk - mu) ** 2))
        blocks.append((blk - mu) / (sd + _EPS_GN))
    gx = jnp.concatenate(blocks, axis=0) * gnw + gnb
    info = jax.nn.sigmoid(gx * wg) >= 0.5
    x1 = jnp.where(info, gx, 0.0)
    x2 = gx - x1
    h = _C // 2
    up = _mm(sq1, x1[:h] + x2[h:])
    low = _mm(sq2, x1[h:] + x2[:h])
    pad_ref[...] = jnp.zeros_like(pad_ref)
    pad_ref[:, 33:33 + _N] = up
    acc = jnp.zeros((_C, _N), jnp.float32)
    k = 0
    for dy in (-1, 0, 1):
        for dx in (-1, 0, 1):
            off = 33 + 32 * dy + dx
            xs = pad_ref[:, off:off + _N]
            if dx == -1:
                xs = jnp.where(wmod != 0, xs, 0.0)
            elif dx == 1:
                xs = jnp.where(wmod != 31, xs, 0.0)
            acc = acc + _mm(w9_ref[k], xs)
            k += 1
    y1c = acc + gwcb + _mm(pwc1, up)
    y2c = jnp.concatenate([_mm(pwc2, low), low], axis=0)
    mcat = jnp.concatenate([jnp.mean(y1c, axis=1, keepdims=True),
                            jnp.mean(y2c, axis=1, keepdims=True)], axis=0)
    e = jnp.exp(mcat - jnp.max(mcat))
    att = e / jnp.sum(e)
    return att[:_C] * y1c + att[_C:] * y2c


def _post_fc1_body(in_ref, bm_ref, bd_ref, bg_ref, bb_ref, res_ref, gnw_ref,
                   gnb_ref, wg_ref, sq1_ref, sq2_ref, w9_ref, gwcb_ref,
                   pwc1_ref, pwc2_ref, f1w_ref, f1b_ref, t_ref, y2_ref,
                   st_ref, pad_ref):
    b = pl.program_id(0)
    wmod = jax.lax.broadcasted_iota(jnp.int32, (1, _N), 1) % 32
    xx = ((in_ref[0] - bm_ref[...]) * jax.lax.rsqrt(bd_ref[...] + _EPS_BN)
          * bg_ref[...] + bb_ref[...] + res_ref[0])
    so = _scconv(xx, gnw_ref[...], gnb_ref[...], wg_ref[...], sq1_ref[...],
                 sq2_ref[...], w9_ref, gwcb_ref[...], pwc1_ref[...],
                 pwc2_ref[...], pad_ref, wmod)
    mu = jnp.mean(so, axis=1, keepdims=True)
    va = jnp.mean((so - mu) ** 2, axis=1, keepdims=True)
    t = jnp.maximum((so - mu) * jax.lax.rsqrt(va + _EPS_IN), 0.0)
    t_ref[0] = t
    y2 = _mm(f1w_ref[...], t) + f1b_ref[...]
    y2_ref[0] = y2

    @pl.when(b == 0)
    def _():
        st_ref[...] = jnp.zeros_like(st_ref)

    st_ref[0, :, 0:1] = st_ref[0, :, 0:1] + jnp.sum(y2, axis=1, keepdims=True)
    st_ref[0, :, 1:2] = st_ref[0, :, 1:2] + jnp.sum(y2 * y2, axis=1,
                                                    keepdims=True)


def _post_fc1_call(in_raw, bn_cols, res, gnw, gnb, wg, sq1, sq2, w9, gwcb,
                   pwc1, pwc2, f1w, f1b):
    small = lambda shape: pl.BlockSpec(shape, lambda b: (0,) * len(shape))
    return pl.pallas_call(
        _post_fc1_body,
        grid=(_B,),
        in_specs=[
            pl.BlockSpec((1, _C, _N), lambda b: (b, 0, 0)),
            small((_C, 1)), small((_C, 1)), small((_C, 1)), small((_C, 1)),
            pl.BlockSpec((1, _C, _N), lambda b: (b, 0, 0)),
            small((_C, 1)), small((_C, 1)), small((_C, 1)),
            small((24, 48)), small((24, 48)),
            small((9, _C, 24)), small((_C, 1)),
            small((_C, 24)), small((_C - 24, 24)),
            small((_C, _C)), small((_C, 1)),
        ],
        out_specs=[
            pl.BlockSpec((1, _C, _N), lambda b: (b, 0, 0)),
            pl.BlockSpec((1, _C, _N), lambda b: (b, 0, 0)),
            pl.BlockSpec((1, _C, 8), lambda b: (0, 0, 0)),
        ],
        out_shape=[
            jax.ShapeDtypeStruct((_B, _C, _N), jnp.float32),
            jax.ShapeDtypeStruct((_B, _C, _N), jnp.float32),
            jax.ShapeDtypeStruct((1, _C, 8), jnp.float32),
        ],
        scratch_shapes=[pltpu.VMEM((24, 1152), jnp.float32)],
    )(in_raw, *bn_cols, res, gnw, gnb, wg, sq1, sq2, w9, gwcb, pwc1, pwc2,
      f1w, f1b)


def _post_final_body(in_ref, bm_ref, bd_ref, bg_ref, bb_ref, res_ref,
                     gnw_ref, gnb_ref, wg_ref, sq1_ref, sq2_ref, w9_ref,
                     gwcb_ref, pwc1_ref, pwc2_ref, x_ref, fin_ref, pad_ref):
    wmod = jax.lax.broadcasted_iota(jnp.int32, (1, _N), 1) % 32
    xx = ((in_ref[0] - bm_ref[...]) * jax.lax.rsqrt(bd_ref[...] + _EPS_BN)
          * bg_ref[...] + bb_ref[...] + res_ref[0])
    so = _scconv(xx, gnw_ref[...], gnb_ref[...], wg_ref[...], sq1_ref[...],
                 sq2_ref[...], w9_ref, gwcb_ref[...], pwc1_ref[...],
                 pwc2_ref[...], pad_ref, wmod)
    mu = jnp.mean(so, axis=1, keepdims=True)
    va = jnp.mean((so - mu) ** 2, axis=1, keepdims=True)
    fin_ref[0] = (so - mu) * jax.lax.rsqrt(va + _EPS_IN) + x_ref[0]


def _post_final_call(in_raw, bn_cols, res, gnw, gnb, wg, sq1, sq2, w9, gwcb,
                     pwc1, pwc2, x):
    small = lambda shape: pl.BlockSpec(shape, lambda b: (0,) * len(shape))
    return pl.pallas_call(
        _post_final_body,
        grid=(_B,),
        in_specs=[
            pl.BlockSpec((1, _C, _N), lambda b: (b, 0, 0)),
            small((_C, 1)), small((_C, 1)), small((_C, 1)), small((_C, 1)),
            pl.BlockSpec((1, _C, _N), lambda b: (b, 0, 0)),
            small((_C, 1)), small((_C, 1)), small((_C, 1)),
            small((24, 48)), small((24, 48)),
            small((9, _C, 24)), small((_C, 1)),
            small((_C, 24)), small((_C - 24, 24)),
            pl.BlockSpec((1, _C, _N), lambda b: (b, 0, 0)),
        ],
        out_specs=[pl.BlockSpec((1, _C, _N), lambda b: (b, 0, 0))],
        out_shape=[jax.ShapeDtypeStruct((_B, _C, _N), jnp.float32)],
        scratch_shapes=[pltpu.VMEM((24, 1152), jnp.float32)],
    )(in_raw, *bn_cols, res, gnw, gnb, wg, sq1, sq2, w9, gwcb, pwc1, pwc2, x)


def _bn_cols(st, g, b):
    cnt = float(_B * _N)
    mean = st[0, :, 0] / cnt
    var = st[0, :, 1] / cnt - mean * mean
    return mean[:, None], var[:, None], g[:, None], b[:, None]


def _bn_two_pass(y, acc_sum, g, b):
    cnt = float(_B * _N)
    mean = jnp.sum(acc_sum[0, :, 0:128], axis=1) / cnt
    var = jnp.sum(_var_call(y, mean[:, None])[0, :, 0:128], axis=1) / cnt
    return mean[:, None], var[:, None], g[:, None], b[:, None]


def _mr_split(mw):
    gi = jnp.arange(_C2) // 48

    def pick(j):
        valid = (j[None, :] // 48) == gi[:, None]
        q = jnp.clip(j[None, :] - 48 * gi[:, None], 0, 47)
        return jnp.where(valid, jnp.take_along_axis(mw, q, axis=1), 0.0)

    je = 2 * jnp.arange(_C)
    return pick(je), pick(je + 1)


def _gwc9(wc):
    wt = jnp.transpose(wc, (2, 3, 0, 1))
    z = jnp.zeros((3, 3, 48, 12), jnp.float32)
    top = jnp.concatenate([wt[:, :, :48, :], z], axis=3)
    bot = jnp.concatenate([z, wt[:, :, 48:, :]], axis=3)
    return jnp.concatenate([top, bot], axis=2).reshape(9, _C, 24)


def kernel(x, g1_fc1_w, g1_fc1_b, g1_bn1_g, g1_bn1_b, g1_rel, g1_mr_w,
           g1_mr_b, g1_fc2_w, g1_fc2_b, g1_bn2_g, g1_bn2_b,
           g2_fc1_w, g2_fc1_b, g2_bn1_g, g2_bn1_b, g2_rel, g2_mr_w,
           g2_mr_b, g2_fc2_w, g2_fc2_b, g2_bn2_g, g2_bn2_b,
           s1_gn_w, s1_gn_b, s1_sq1_w, s1_sq2_w, s1_gwc_w, s1_gwc_b,
           s1_pwc1_w, s1_pwc2_w,
           s2_gn_w, s2_gn_b, s2_sq1_w, s2_sq2_w, s2_gwc_w, s2_gwc_b,
           s2_pwc1_w, s2_pwc2_w):
    col = lambda v: v.reshape(-1, 1)
    xf = x.reshape(_B, _C, _N)

    y1r, acc1, _st1 = _fc1_call(xf, g1_fc1_w, col(g1_fc1_b))
    bn1 = _bn_two_pass(y1r, acc1, g1_bn1_g, g1_bn1_b)
    wa1, wb1 = _mr_split(g1_mr_w)
    o2r, st2 = _grapher_call(y1r, bn1, g1_rel.reshape(_N, _N), wa1,
                             wb1, col(g1_mr_b), g1_fc2_w, col(g1_fc2_b))
    bn2 = _bn_cols(st2, g1_bn2_g, g1_bn2_b)

    t, y2r, st3 = _post_fc1_call(
        o2r, bn2, xf, col(s1_gn_w), col(s1_gn_b),
        col(s1_gn_w / jnp.sum(s1_gn_w)), s1_sq1_w, s1_sq2_w, _gwc9(s1_gwc_w),
        col(s1_gwc_b), s1_pwc1_w, s1_pwc2_w, g2_fc1_w, col(g2_fc1_b))
    bn3 = _bn_cols(st3, g2_bn1_g, g2_bn1_b)
    wa2, wb2 = _mr_split(g2_mr_w)
    o4r, st4 = _grapher_call(y2r, bn3, g2_rel.reshape(_N, _N), wa2,
                             wb2, col(g2_mr_b), g2_fc2_w, col(g2_fc2_b))
    bn4 = _bn_cols(st4, g2_bn2_g, g2_bn2_b)

    fin = _post_final_call(
        o4r, bn4, t, col(s2_gn_w), col(s2_gn_b),
        col(s2_gn_w / jnp.sum(s2_gn_w)), s2_sq1_w, s2_sq2_w, _gwc9(s2_gwc_w),
        col(s2_gwc_b), s2_pwc1_w, s2_pwc2_w, xf)
    return fin[0].reshape(_B, _C, _H, _W)


# speedup vs baseline: 270.2888x; 1.0076x over previous
import jax
import jax.numpy as jnp
from jax.experimental import pallas as pl
from jax.experimental.pallas import tpu as pltpu

_B, _C, _H, _W = 16, 96, 32, 32
_N = _H * _W
_K = 7
_C2 = 2 * _C
_EPS_BN = 1e-5
_EPS_IN = 1e-5
_EPS_GN = 1e-10


def _mm(a, b):
    return jax.lax.dot_general(a.astype(jnp.bfloat16), b.astype(jnp.bfloat16),
                               (((1,), (0,)), ((), ())),
                               preferred_element_type=jnp.float32)


def _fc1_body(x_ref, w_ref, bcol_ref, y_ref, acc_ref, st_ref):
    b = pl.program_id(0)
    y = _mm(w_ref[...], x_ref[0]) + bcol_ref[...]
    y_ref[0] = y

    @pl.when(b == 0)
    def _():
        acc_ref[...] = jnp.zeros_like(acc_ref)

    a = acc_ref[0, :, 0:128]
    for k in range(_N // 128):
        a = a + y[:, 128 * k:128 * (k + 1)]
    acc_ref[0, :, 0:128] = a

    @pl.when(b == _B - 1)
    def _():
        st_ref[...] = jnp.zeros_like(st_ref)
        st_ref[0, :, 0:1] = jnp.sum(acc_ref[0, :, 0:128], axis=1,
                                    keepdims=True)


def _fc1_call(x, w, bcol):
    return pl.pallas_call(
        _fc1_body,
        grid=(_B,),
        in_specs=[
            pl.BlockSpec((1, _C, _N), lambda b: (b, 0, 0)),
            pl.BlockSpec((_C, _C), lambda b: (0, 0)),
            pl.BlockSpec((_C, 1), lambda b: (0, 0)),
        ],
        out_specs=[
            pl.BlockSpec((1, _C, _N), lambda b: (b, 0, 0)),
            pl.BlockSpec((1, _C, _N), lambda b: (0, 0, 0)),
            pl.BlockSpec((1, _C, 8), lambda b: (0, 0, 0)),
        ],
        out_shape=[
            jax.ShapeDtypeStruct((_B, _C, _N), jnp.float32),
            jax.ShapeDtypeStruct((1, _C, _N), jnp.float32),
            jax.ShapeDtypeStruct((1, _C, 8), jnp.float32),
        ],
    )(x, w, bcol)


def _var_body(y_ref, m_ref, acc_ref, st_ref):
    b = pl.program_id(0)
    t = y_ref[0] - m_ref[...]
    t = t * t

    @pl.when(b == 0)
    def _():
        acc_ref[...] = jnp.zeros_like(acc_ref)

    a = acc_ref[0, :, 0:128]
    for k in range(_N // 128):
        a = a + t[:, 128 * k:128 * (k + 1)]
    acc_ref[0, :, 0:128] = a

    @pl.when(b == _B - 1)
    def _():
        st_ref[...] = jnp.zeros_like(st_ref)
        st_ref[0, :, 0:1] = jnp.sum(acc_ref[0, :, 0:128], axis=1,
                                    keepdims=True)


def _var_call(y, mcol):
    return pl.pallas_call(
        _var_body,
        grid=(_B,),
        in_specs=[
            pl.BlockSpec((1, _C, _N), lambda b: (b, 0, 0)),
            pl.BlockSpec((_C, 1), lambda b: (0, 0)),
        ],
        out_specs=[
            pl.BlockSpec((1, _C, _N), lambda b: (0, 0, 0)),
            pl.BlockSpec((1, _C, 8), lambda b: (0, 0, 0)),
        ],
        out_shape=[
            jax.ShapeDtypeStruct((1, _C, _N), jnp.float32),
            jax.ShapeDtypeStruct((1, _C, 8), jnp.float32),
        ],
    )(y, mcol)[0]


def _grapher_body(y_ref, bm_ref, bd_ref, bg_ref, bb_ref, rel_ref, wa_ref,
                  wb_ref, mb_ref, w2_ref, b2_ref, out_ref, st_ref):
    b = pl.program_id(0)
    y = ((y_ref[0] - bm_ref[...]) * jax.lax.rsqrt(bd_ref[...] + _EPS_BN)
         * bg_ref[...] + bb_ref[...])
    xn = y / jnp.maximum(
        jnp.sqrt(jnp.sum(y * y, axis=0, keepdims=True)), 1e-12)
    xnt = jnp.transpose(xn)
    sq_col = jnp.sum(xnt * xnt, axis=1, keepdims=True)
    sq_row = jnp.transpose(sq_col)
    cur = sq_col - 2.0 * _mm(xnt, xn) + sq_row + rel_ref[...]
    iota_l = jax.lax.broadcasted_iota(jnp.int32, (_N, _N), 1)
    y1s = y.astype(jnp.bfloat16)
    r1 = y - y1s.astype(jnp.float32)
    y2s = r1.astype(jnp.bfloat16)
    y3s = (r1 - y2s.astype(jnp.float32)).astype(jnp.bfloat16)
    dmax = jnp.full((_C, _N), -jnp.inf, dtype=jnp.float32)
    for _ in range(_K):
        minv = jnp.min(cur, axis=1, keepdims=True)
        idx = jnp.min(jnp.where(cur == minv, iota_l, _N), axis=1, keepdims=True)
        sel = iota_l == idx
        cur = jnp.where(sel, jnp.inf, cur)
        oh = jnp.where(sel, 1.0, 0.0).astype(jnp.bfloat16)
        nt = (((1,), (1,)), ((), ()))
        g = (jax.lax.dot_general(y1s, oh, nt, preferred_element_type=jnp.float32)
             + jax.lax.dot_general(y2s, oh, nt, preferred_element_type=jnp.float32)
             + jax.lax.dot_general(y3s, oh, nt, preferred_element_type=jnp.float32))
        dmax = jnp.maximum(dmax, g)
    d = dmax - y
    m = _mm(wa_ref[...], y) + _mm(wb_ref[...], d) + mb_ref[...]
    mu = jnp.mean(m, axis=1, keepdims=True)
    va = jnp.mean((m - mu) ** 2, axis=1, keepdims=True)
    m = jax.nn.gelu((m - mu) * jax.lax.rsqrt(va + _EPS_IN))
    out = _mm(w2_ref[...], m) + b2_ref[...]
    out_ref[0] = out

    @pl.when(b == 0)
    def _():
        st_ref[...] = jnp.zeros_like(st_ref)

    st_ref[0, :, 0:1] = st_ref[0, :, 0:1] + jnp.sum(out, axis=1, keepdims=True)
    st_ref[0, :, 1:2] = st_ref[0, :, 1:2] + jnp.sum(out * out, axis=1,
                                                    keepdims=True)


def _grapher_call(y_raw, bn_cols, rel, wa, wb, mb, w2, b2):
    return pl.pallas_call(
        _grapher_body,
        grid=(_B,),
        in_specs=[
            pl.BlockSpec((1, _C, _N), lambda b: (b, 0, 0)),
            pl.BlockSpec((_C, 1), lambda b: (0, 0)),
            pl.BlockSpec((_C, 1), lambda b: (0, 0)),
            pl.BlockSpec((_C, 1), lambda b: (0, 0)),
            pl.BlockSpec((_C, 1), lambda b: (0, 0)),
            pl.BlockSpec((_N, _N), lambda b: (0, 0)),
            pl.BlockSpec((_C2, _C), lambda b: (0, 0)),
            pl.BlockSpec((_C2, _C), lambda b: (0, 0)),
            pl.BlockSpec((_C2, 1), lambda b: (0, 0)),
            pl.BlockSpec((_C, _C2), lambda b: (0, 0)),
            pl.BlockSpec((_C, 1), lambda b: (0, 0)),
        ],
        out_specs=[
            pl.BlockSpec((1, _C, _N), lambda b: (b, 0, 0)),
            pl.BlockSpec((1, _C, 8), lambda b: (0, 0, 0)),
        ],
        out_shape=[
            jax.ShapeDtypeStruct((_B, _C, _N), jnp.float32),
            jax.ShapeDtypeStruct((1, _C, 8), jnp.float32),
        ],
    )(y_raw, *bn_cols, rel, wa, wb, mb, w2, b2)


def _scconv(xx, gnw, gnb, wg, sq1, sq2, w9_ref, gwcb, pwc1, pwc2, pad_ref,
            wmod):
    blocks = []
    for g in range(4):
        blk = xx[24 * g:24 * (g + 1), :]
        mu = jnp.mean(blk)
        sd = jnp.sqrt(jnp.mean((blk - mu) ** 2))
        blocks.append((blk - mu) / (sd + _EPS_GN))
    gx = jnp.concatenate(blocks, axis=0) * gnw + gnb
    info = jax.nn.sigmoid(gx * wg) >= 0.5
    x1 = jnp.where(info, gx, 0.0)
    x2 = gx - x1
    h = _C // 2
    up = _mm(sq1, x1[:h] + x2[h:])
    low = _mm(sq2, x1[h:] + x2[:h])
    pad_ref[...] = jnp.zeros_like(pad_ref)
    pad_ref[:, 33:33 + _N] = up
    acc = jnp.zeros((_C, _N), jnp.float32)
    k = 0
    for dy in (-1, 0, 1):
        for dx in (-1, 0, 1):
            off = 33 + 32 * dy + dx
            xs = pad_ref[:, off:off + _N]
            if dx == -1:
                xs = jnp.where(wmod != 0, xs, 0.0)
            elif dx == 1:
                xs = jnp.where(wmod != 31, xs, 0.0)
            acc = acc + _mm(w9_ref[k], xs)
            k += 1
    y1c = acc + gwcb + _mm(pwc1, up)
    y2c = jnp.concatenate([_mm(pwc2, low), low], axis=0)
    mcat = jnp.concatenate([jnp.mean(y1c, axis=1, keepdims=True),
                            jnp.mean(y2c, axis=1, keepdims=True)], axis=0)
    e = jnp.exp(mcat - jnp.max(mcat))
    att = e / jnp.sum(e)
    return att[:_C] * y1c + att[_C:] * y2c


def _post_fc1_body(in_ref, bm_ref, bd_ref, bg_ref, bb_ref, res_ref, gnw_ref,
                   gnb_ref, wg_ref, sq1_ref, sq2_ref, w9_ref, gwcb_ref,
                   pwc1_ref, pwc2_ref, f1w_ref, f1b_ref, t_ref, y2_ref,
                   st_ref, pad_ref):
    b = pl.program_id(0)
    wmod = jax.lax.broadcasted_iota(jnp.int32, (1, _N), 1) % 32
    xx = ((in_ref[0] - bm_ref[...]) * jax.lax.rsqrt(bd_ref[...] + _EPS_BN)
          * bg_ref[...] + bb_ref[...] + res_ref[0])
    so = _scconv(xx, gnw_ref[...], gnb_ref[...], wg_ref[...], sq1_ref[...],
                 sq2_ref[...], w9_ref, gwcb_ref[...], pwc1_ref[...],
                 pwc2_ref[...], pad_ref, wmod)
    mu = jnp.mean(so, axis=1, keepdims=True)
    va = jnp.mean((so - mu) ** 2, axis=1, keepdims=True)
    t = jnp.maximum((so - mu) * jax.lax.rsqrt(va + _EPS_IN), 0.0)
    t_ref[0] = t
    y2 = _mm(f1w_ref[...], t) + f1b_ref[...]
    y2_ref[0] = y2

    @pl.when(b == 0)
    def _():
        st_ref[...] = jnp.zeros_like(st_ref)

    st_ref[0, :, 0:1] = st_ref[0, :, 0:1] + jnp.sum(y2, axis=1, keepdims=True)
    st_ref[0, :, 1:2] = st_ref[0, :, 1:2] + jnp.sum(y2 * y2, axis=1,
                                                    keepdims=True)


def _post_fc1_call(in_raw, bn_cols, res, gnw, gnb, wg, sq1, sq2, w9, gwcb,
                   pwc1, pwc2, f1w, f1b):
    small = lambda shape: pl.BlockSpec(shape, lambda b: (0,) * len(shape))
    return pl.pallas_call(
        _post_fc1_body,
        grid=(_B,),
        in_specs=[
            pl.BlockSpec((1, _C, _N), lambda b: (b, 0, 0)),
            small((_C, 1)), small((_C, 1)), small((_C, 1)), small((_C, 1)),
            pl.BlockSpec((1, _C, _N), lambda b: (b, 0, 0)),
            small((_C, 1)), small((_C, 1)), small((_C, 1)),
            small((24, 48)), small((24, 48)),
            small((9, _C, 24)), small((_C, 1)),
            small((_C, 24)), small((_C - 24, 24)),
            small((_C, _C)), small((_C, 1)),
        ],
        out_specs=[
            pl.BlockSpec((1, _C, _N), lambda b: (b, 0, 0)),
            pl.BlockSpec((1, _C, _N), lambda b: (b, 0, 0)),
            pl.BlockSpec((1, _C, 8), lambda b: (0, 0, 0)),
        ],
        out_shape=[
            jax.ShapeDtypeStruct((_B, _C, _N), jnp.float32),
            jax.ShapeDtypeStruct((_B, _C, _N), jnp.float32),
            jax.ShapeDtypeStruct((1, _C, 8), jnp.float32),
        ],
        scratch_shapes=[pltpu.VMEM((24, 1152), jnp.float32)],
    )(in_raw, *bn_cols, res, gnw, gnb, wg, sq1, sq2, w9, gwcb, pwc1, pwc2,
      f1w, f1b)


def _post_final_body(in_ref, bm_ref, bd_ref, bg_ref, bb_ref, res_ref,
                     gnw_ref, gnb_ref, wg_ref, sq1_ref, sq2_ref, w9_ref,
                     gwcb_ref, pwc1_ref, pwc2_ref, x_ref, fin_ref, pad_ref):
    wmod = jax.lax.broadcasted_iota(jnp.int32, (1, _N), 1) % 32
    xx = ((in_ref[0] - bm_ref[...]) * jax.lax.rsqrt(bd_ref[...] + _EPS_BN)
          * bg_ref[...] + bb_ref[...] + res_ref[0])
    so = _scconv(xx, gnw_ref[...], gnb_ref[...], wg_ref[...], sq1_ref[...],
                 sq2_ref[...], w9_ref, gwcb_ref[...], pwc1_ref[...],
                 pwc2_ref[...], pad_ref, wmod)
    mu = jnp.mean(so, axis=1, keepdims=True)
    va = jnp.mean((so - mu) ** 2, axis=1, keepdims=True)
    fin_ref[0] = (so - mu) * jax.lax.rsqrt(va + _EPS_IN) + x_ref[0]


def _post_final_call(in_raw, bn_cols, res, gnw, gnb, wg, sq1, sq2, w9, gwcb,
                     pwc1, pwc2, x):
    small = lambda shape: pl.BlockSpec(shape, lambda b: (0,) * len(shape))
    return pl.pallas_call(
        _post_final_body,
        grid=(_B,),
        in_specs=[
            pl.BlockSpec((1, _C, _N), lambda b: (b, 0, 0)),
            small((_C, 1)), small((_C, 1)), small((_C, 1)), small((_C, 1)),
            pl.BlockSpec((1, _C, _N), lambda b: (b, 0, 0)),
            small((_C, 1)), small((_C, 1)), small((_C, 1)),
            small((24, 48)), small((24, 48)),
            small((9, _C, 24)), small((_C, 1)),
            small((_C, 24)), small((_C - 24, 24)),
            pl.BlockSpec((1, _C, _N), lambda b: (b, 0, 0)),
        ],
        out_specs=[pl.BlockSpec((1, _C, _N), lambda b: (b, 0, 0))],
        out_shape=[jax.ShapeDtypeStruct((_B, _C, _N), jnp.float32)],
        scratch_shapes=[pltpu.VMEM((24, 1152), jnp.float32)],
    )(in_raw, *bn_cols, res, gnw, gnb, wg, sq1, sq2, w9, gwcb, pwc1, pwc2, x)


def _bn_cols(st, g, b):
    cnt = float(_B * _N)
    mean = st[0, :, 0] / cnt
    var = st[0, :, 1] / cnt - mean * mean
    return mean[:, None], var[:, None], g[:, None], b[:, None]


def _bn_two_pass(y, acc_sum, g, b):
    cnt = float(_B * _N)
    mean = jnp.sum(acc_sum[0, :, 0:128], axis=1) / cnt
    var = jnp.sum(_var_call(y, mean[:, None])[0, :, 0:128], axis=1) / cnt
    return mean[:, None], var[:, None], g[:, None], b[:, None]


def _stats_body(y_ref, st_ref):
    acc = jnp.zeros((_C, 128), jnp.float32)
    for k in range(_N // 128):
        for b in range(_B):
            acc = acc + y_ref[b, :, 128 * k:128 * (k + 1)]
    mean = jnp.sum(acc, axis=1, keepdims=True) * (1.0 / float(_B * _N))
    acc2 = jnp.zeros((_C, 128), jnp.float32)
    for k in range(_N // 128):
        for b in range(_B):
            t = y_ref[b, :, 128 * k:128 * (k + 1)] - mean
            acc2 = acc2 + t * t
    st_ref[...] = jnp.zeros_like(st_ref)
    st_ref[0, :, 0:1] = mean
    st_ref[0, :, 1:2] = jnp.sum(acc2, axis=1, keepdims=True) * (
        1.0 / float(_B * _N))


def _bn_stats_call(y, g, b):
    st = pl.pallas_call(
        _stats_body,
        out_shape=jax.ShapeDtypeStruct((1, _C, 8), jnp.float32),
    )(y)
    return (st[0, :, 0:1], st[0, :, 1:2], g[:, None], b[:, None])


def _mr_split(mw):
    gi = jnp.arange(_C2) // 48

    def pick(j):
        valid = (j[None, :] // 48) == gi[:, None]
        q = jnp.clip(j[None, :] - 48 * gi[:, None], 0, 47)
        return jnp.where(valid, jnp.take_along_axis(mw, q, axis=1), 0.0)

    je = 2 * jnp.arange(_C)
    return pick(je), pick(je + 1)


def _gwc9(wc):
    wt = jnp.transpose(wc, (2, 3, 0, 1))
    z = jnp.zeros((3, 3, 48, 12), jnp.float32)
    top = jnp.concatenate([wt[:, :, :48, :], z], axis=3)
    bot = jnp.concatenate([z, wt[:, :, 48:, :]], axis=3)
    return jnp.concatenate([top, bot], axis=2).reshape(9, _C, 24)


def kernel(x, g1_fc1_w, g1_fc1_b, g1_bn1_g, g1_bn1_b, g1_rel, g1_mr_w,
           g1_mr_b, g1_fc2_w, g1_fc2_b, g1_bn2_g, g1_bn2_b,
           g2_fc1_w, g2_fc1_b, g2_bn1_g, g2_bn1_b, g2_rel, g2_mr_w,
           g2_mr_b, g2_fc2_w, g2_fc2_b, g2_bn2_g, g2_bn2_b,
           s1_gn_w, s1_gn_b, s1_sq1_w, s1_sq2_w, s1_gwc_w, s1_gwc_b,
           s1_pwc1_w, s1_pwc2_w,
           s2_gn_w, s2_gn_b, s2_sq1_w, s2_sq2_w, s2_gwc_w, s2_gwc_b,
           s2_pwc1_w, s2_pwc2_w):
    col = lambda v: v.reshape(-1, 1)
    xf = x.reshape(_B, _C, _N)

    y1r, acc1, _st1 = _fc1_call(xf, g1_fc1_w, col(g1_fc1_b))
    bn1 = _bn_two_pass(y1r, acc1, g1_bn1_g, g1_bn1_b)
    wa1, wb1 = _mr_split(g1_mr_w)
    o2r, st2 = _grapher_call(y1r, bn1, g1_rel.reshape(_N, _N), wa1,
                             wb1, col(g1_mr_b), g1_fc2_w, col(g1_fc2_b))
    bn2 = _bn_cols(st2, g1_bn2_g, g1_bn2_b)

    t, y2r, st3 = _post_fc1_call(
        o2r, bn2, xf, col(s1_gn_w), col(s1_gn_b),
        col(s1_gn_w / jnp.sum(s1_gn_w)), s1_sq1_w, s1_sq2_w, _gwc9(s1_gwc_w),
        col(s1_gwc_b), s1_pwc1_w, s1_pwc2_w, g2_fc1_w, col(g2_fc1_b))
    bn3 = _bn_cols(st3, g2_bn1_g, g2_bn1_b)
    wa2, wb2 = _mr_split(g2_mr_w)
    o4r, st4 = _grapher_call(y2r, bn3, g2_rel.reshape(_N, _N), wa2,
                             wb2, col(g2_mr_b), g2_fc2_w, col(g2_fc2_b))
    bn4 = _bn_cols(st4, g2_bn2_g, g2_bn2_b)

    fin = _post_final_call(
        o4r, bn4, t, col(s2_gn_w), col(s2_gn_b),
        col(s2_gn_w / jnp.sum(s2_gn_w)), s2_sq1_w, s2_sq2_w, _gwc9(s2_gwc_w),
        col(s2_gwc_b), s2_pwc1_w, s2_pwc2_w, xf)
    return fin[0].reshape(_B, _C, _H, _W)
